# baseline (speedup 1.0000x reference)
import functools

import jax
import jax.numpy as jnp
from jax import lax
from jax.experimental import pallas as pl
from jax.experimental.pallas import tpu as pltpu
from jax.experimental.pallas import tpu_sc as plsc

D = 128
R = 1024
CH = 64
LANES = 16
NB = 2


def _dense_in_body(x_ref, wr_ref, b_ref, wo_ref, y_ref, r_ref):
    xb = x_ref[...]
    y_ref[...] = jnp.dot(xb, wr_ref[...], preferred_element_type=jnp.float32)
    r_ref[...] = jnp.dot(xb, wo_ref[...], preferred_element_type=jnp.float32) + b_ref[...]


def _dense_mid_body(n_real, acc_ref, r1_ref, wr_ref, b_ref, wo_ref, y2_ref, r2_ref):
    h = jnp.maximum(acc_ref[0] + acc_ref[1] + r1_ref[...], 0.0)
    rows = pl.program_id(0) * R + lax.broadcasted_iota(jnp.int32, (R, 1), 0)
    h = jnp.where(rows < n_real, h, 0.0)
    y2_ref[...] = jnp.dot(h, wr_ref[...], preferred_element_type=jnp.float32)
    r2_ref[...] = jnp.dot(h, wo_ref[...], preferred_element_type=jnp.float32) + b_ref[...]


def _dense_out_body(acc_ref, r2_ref, o_ref):
    o_ref[...] = acc_ref[0] + acc_ref[1] + r2_ref[...]


def _row_spec():
    return pl.BlockSpec((R, D), lambda i: (i, 0))


def _w_spec():
    return pl.BlockSpec((D, D), lambda i: (0, 0))


def _b_spec():
    return pl.BlockSpec((1, D), lambda i: (0, 0))


def _acc_spec():
    return pl.BlockSpec((2, R, D), lambda i: (0, i, 0))


def _dense_in(x, wr, b, wo, grid_n):
    n = x.shape[0]
    return pl.pallas_call(
        _dense_in_body,
        grid=(grid_n,),
        in_specs=[_row_spec(), _w_spec(), _b_spec(), _w_spec()],
        out_specs=[_row_spec(), _row_spec()],
        out_shape=[jax.ShapeDtypeStruct((n, D), jnp.float32)] * 2,
    )(x, wr, b, wo)


def _dense_mid(acc, r1, wr, b, wo, grid_n, n):
    return pl.pallas_call(
        functools.partial(_dense_mid_body, n),
        grid=(grid_n,),
        in_specs=[_acc_spec(), _row_spec(), _w_spec(), _b_spec(), _w_spec()],
        out_specs=[_row_spec(), _row_spec()],
        out_shape=[jax.ShapeDtypeStruct((n, D), jnp.float32)] * 2,
    )(acc, r1, wr, b, wo)


def _dense_out(acc, r2, grid_n, n):
    return pl.pallas_call(
        _dense_out_body,
        grid=(grid_n,),
        in_specs=[_acc_spec(), _row_spec()],
        out_specs=_row_spec(),
        out_shape=jax.ShapeDtypeStruct((n, D), jnp.float32),
    )(acc, r2)


@functools.lru_cache(maxsize=None)
def _make_segsum(np_, ntot, nc, ns):
    rpt = np_ // ns
    nw = nc * ns
    assert ntot % 8 == 0
    q8 = (ntot // nw) // 8 * 8
    kp = 0
    for cand in range(min(q8, 96), 0, -8):
        if cand % NB == 0 and 2 * cand * CH * 4 + NB * CH * D * 4 <= 180 * 1024:
            kp = cand
            break
    assert kp and q8 % NB == 0 and kp % NB == 0, (q8, kp, NB)
    nfull, remph = divmod(q8, kp)
    xrounds = []
    left, start = ntot - nw * q8, nw * q8
    while left:
        m = min(nw, left // 8)
        xrounds.append((start, m))
        start += m * 8
        left -= m * 8
    mesh = plsc.VectorSubcoreMesh(core_axis_name="c", subcore_axis_name="s")

    @functools.partial(
        pl.kernel,
        mesh=mesh,
        out_type=jax.ShapeDtypeStruct((nc, np_, D), jnp.float32),
        scratch_types=[
            pltpu.VMEM((kp, CH), jnp.int32),
            pltpu.VMEM((kp, CH), jnp.int32),
            pltpu.VMEM((NB, CH, D), jnp.float32),
            pltpu.VMEM_SHARED((np_, D), jnp.float32),
            pltpu.SemaphoreType.DMA((NB,)),
            pltpu.SemaphoreType.DMA,
            pltpu.SemaphoreType.DMA,
        ],
    )
    def seg(y_hbm, ei_hbm, out_hbm, src_v, dst_v, rows_v, acc_sh,
            sem, isem1, isem2):
        c = lax.axis_index("c")
        s = lax.axis_index("s")
        wid = c * ns + s
        off0 = wid * q8

        icp1 = pltpu.async_copy(ei_hbm.at[0, pl.ds(off0, kp)], src_v, isem1)
        icp2 = pltpu.async_copy(ei_hbm.at[1, pl.ds(off0, kp)], dst_v, isem2)

        def zrow(i, carry):
            for j in range(D // LANES):
                rows_v[0, i, pl.ds(j * LANES, LANES)] = jnp.zeros((LANES,), jnp.float32)
            return carry
        lax.fori_loop(0, CH, zrow, 0)
        for t in range(rpt // CH):
            pltpu.sync_copy(rows_v.at[0], acc_sh.at[pl.ds(s * rpt + t * CH, CH)])
        icp1.wait()
        icp2.wait()

        for b in range(NB):
            pltpu.async_copy(y_hbm.at[src_v.at[b]], rows_v.at[b], sem.at[b])
        plsc.subcore_barrier()

        def outer(t, carry):
            base = t * NB
            for b in range(NB):
                pltpu.make_async_copy(
                    y_hbm.at[src_v.at[base + b]], rows_v.at[b], sem.at[b]).wait()
                pltpu.sync_copy(rows_v.at[b], acc_sh.at[dst_v.at[base + b]], add=True)
                pltpu.async_copy(
                    y_hbm.at[src_v.at[base + NB + b]], rows_v.at[b], sem.at[b])
            return carry

        def run_phase(cnt):
            lax.fori_loop(0, cnt // NB - 1, outer, 0)
            for b in range(NB):
                pltpu.make_async_copy(
                    y_hbm.at[src_v.at[cnt - NB + b]], rows_v.at[b], sem.at[b]).wait()
                pltpu.sync_copy(rows_v.at[b], acc_sh.at[dst_v.at[cnt - NB + b]],
                                add=True)

        def stage_and_run(off, cnt):
            pltpu.sync_copy(ei_hbm.at[0, pl.ds(off, cnt)], src_v.at[pl.ds(0, cnt)])
            pltpu.sync_copy(ei_hbm.at[1, pl.ds(off, cnt)], dst_v.at[pl.ds(0, cnt)])
            for b in range(NB):
                pltpu.async_copy(y_hbm.at[src_v.at[b]], rows_v.at[b], sem.at[b])
            run_phase(cnt)

        run_phase(kp)
        for ph in range(1, nfull):
            stage_and_run(off0 + ph * kp, kp)
        if remph:
            stage_and_run(off0 + nfull * kp, remph)
        for rstart, m in xrounds:
            @pl.when(wid < m)
            def _extra(rstart=rstart):
                stage_and_run(rstart + wid * 8, 8)
        plsc.subcore_barrier()

        for t in range(rpt // CH):
            sl = pl.ds(s * rpt + t * CH, CH)
            pltpu.sync_copy(acc_sh.at[sl], out_hbm.at[c, sl])

    return seg


def kernel(x, edge_index, W1_rel, b1, W1_root, W2_rel, b2, W2_root):
    n, d = x.shape
    e = edge_index.shape[1]
    assert d == D and e % CH == 0
    try:
        info = plsc.get_sparse_core_info()
        nc, ns = info.num_cores, info.num_subcores
    except Exception:
        nc, ns = 2, 16
    align = max(ns * CH, R)
    np_ = -(-n // align) * align
    grid_n = np_ // R
    ntot = e // CH
    ei3 = edge_index.astype(jnp.int32).reshape(2, ntot, CH)
    b1r = b1.reshape(1, D)
    b2r = b2.reshape(1, D)

    seg = _make_segsum(np_, ntot, nc, ns)
    y1, r1 = _dense_in(x, W1_rel, b1r, W1_root, grid_n)
    acc1 = seg(y1, ei3)
    y2, r2 = _dense_mid(acc1, r1, W2_rel, b2r, W2_root, grid_n, n)
    acc2 = seg(y2, ei3)
    return _dense_out(acc2, r2, grid_n, n)


# speedup vs baseline: 1.1448x; 1.1448x over previous
import functools

import jax
import jax.numpy as jnp
from jax import lax
from jax.experimental import pallas as pl
from jax.experimental.pallas import tpu as pltpu
from jax.experimental.pallas import tpu_sc as plsc

D = 128
R = 1024
CH = 64
LANES = 16
NB = 4


def _dense_in_body(x_ref, wr_ref, b_ref, wo_ref, y_ref, r_ref):
    xb = x_ref[...]
    y_ref[...] = jnp.dot(xb, wr_ref[...], preferred_element_type=jnp.float32)
    r_ref[...] = jnp.dot(xb, wo_ref[...], preferred_element_type=jnp.float32) + b_ref[...]


def _dense_mid_body(n_real, acc_ref, r1_ref, wr_ref, b_ref, wo_ref, y2_ref, r2_ref):
    h = jnp.maximum(acc_ref[0] + acc_ref[1] + r1_ref[...], 0.0)
    rows = pl.program_id(0) * R + lax.broadcasted_iota(jnp.int32, (R, 1), 0)
    h = jnp.where(rows < n_real, h, 0.0)
    y2_ref[...] = jnp.dot(h, wr_ref[...], preferred_element_type=jnp.float32)
    r2_ref[...] = jnp.dot(h, wo_ref[...], preferred_element_type=jnp.float32) + b_ref[...]


def _dense_out_body(acc_ref, r2_ref, o_ref):
    o_ref[...] = acc_ref[0] + acc_ref[1] + r2_ref[...]


def _row_spec():
    return pl.BlockSpec((R, D), lambda i: (i, 0))


def _w_spec():
    return pl.BlockSpec((D, D), lambda i: (0, 0))


def _b_spec():
    return pl.BlockSpec((1, D), lambda i: (0, 0))


def _acc_spec():
    return pl.BlockSpec((2, R, D), lambda i: (0, i, 0))


def _dense_in(xp, wr, b, wo, np_):
    return pl.pallas_call(
        _dense_in_body,
        grid=(np_ // R,),
        in_specs=[_row_spec(), _w_spec(), _b_spec(), _w_spec()],
        out_specs=[_row_spec(), _row_spec()],
        out_shape=[jax.ShapeDtypeStruct((np_, D), jnp.float32)] * 2,
    )(xp, wr, b, wo)


def _dense_mid(acc, r1, wr, b, wo, np_, n):
    return pl.pallas_call(
        functools.partial(_dense_mid_body, n),
        grid=(np_ // R,),
        in_specs=[_acc_spec(), _row_spec(), _w_spec(), _b_spec(), _w_spec()],
        out_specs=[_row_spec(), _row_spec()],
        out_shape=[jax.ShapeDtypeStruct((np_, D), jnp.float32)] * 2,
    )(acc, r1, wr, b, wo)


def _dense_out(acc, r2, np_):
    return pl.pallas_call(
        _dense_out_body,
        grid=(np_ // R,),
        in_specs=[_acc_spec(), _row_spec()],
        out_specs=_row_spec(),
        out_shape=jax.ShapeDtypeStruct((np_, D), jnp.float32),
    )(acc, r2)


@functools.lru_cache(maxsize=None)
def _make_segsum(np_, kq, nc, ns):
    rpt = np_ // ns
    kp = kq
    while kp > 48 or kp % NB:
        kp //= 2
    assert kq % kp == 0
    nph = kq // kp
    mesh = plsc.VectorSubcoreMesh(core_axis_name="c", subcore_axis_name="s")

    @functools.partial(
        pl.kernel,
        mesh=mesh,
        out_type=jax.ShapeDtypeStruct((nc, np_, D), jnp.float32),
        scratch_types=[
            pltpu.VMEM((kp, CH), jnp.int32),
            pltpu.VMEM((kp, CH), jnp.int32),
            pltpu.VMEM((NB, CH, D), jnp.float32),
            pltpu.VMEM_SHARED((np_, D), jnp.float32),
            pltpu.SemaphoreType.DMA((NB,)),
            pltpu.SemaphoreType.DMA,
            pltpu.SemaphoreType.DMA,
        ],
    )
    def seg(y_hbm, src_hbm, dst_hbm, out_hbm, src_v, dst_v, rows_v, acc_sh,
            sem, isem1, isem2):
        c = lax.axis_index("c")
        s = lax.axis_index("s")
        wid = c * ns + s
        off0 = wid * kq

        icp1 = pltpu.async_copy(src_hbm.at[pl.ds(off0, kp)], src_v, isem1)
        icp2 = pltpu.async_copy(dst_hbm.at[pl.ds(off0, kp)], dst_v, isem2)

        def zrow(i, carry):
            for j in range(D // LANES):
                rows_v[0, i, pl.ds(j * LANES, LANES)] = jnp.zeros((LANES,), jnp.float32)
            return carry
        lax.fori_loop(0, CH, zrow, 0)
        for t in range(rpt // CH):
            pltpu.sync_copy(rows_v.at[0], acc_sh.at[pl.ds(s * rpt + t * CH, CH)])
        icp1.wait()
        icp2.wait()

        for b in range(NB):
            pltpu.async_copy(y_hbm.at[src_v.at[b]], rows_v.at[b], sem.at[b])
        plsc.subcore_barrier()

        def outer(t, carry):
            base = t * NB
            for b in range(NB):
                pltpu.make_async_copy(
                    y_hbm.at[src_v.at[base + b]], rows_v.at[b], sem.at[b]).wait()
                pltpu.sync_copy(rows_v.at[b], acc_sh.at[dst_v.at[base + b]], add=True)
                pltpu.async_copy(
                    y_hbm.at[src_v.at[base + NB + b]], rows_v.at[b], sem.at[b])
            return carry

        def run_phase():
            lax.fori_loop(0, kp // NB - 1, outer, 0)
            for b in range(NB):
                pltpu.make_async_copy(
                    y_hbm.at[src_v.at[kp - NB + b]], rows_v.at[b], sem.at[b]).wait()
                pltpu.sync_copy(rows_v.at[b], acc_sh.at[dst_v.at[kp - NB + b]],
                                add=True)

        run_phase()
        for ph in range(1, nph):
            off = off0 + ph * kp
            pltpu.sync_copy(src_hbm.at[pl.ds(off, kp)], src_v)
            pltpu.sync_copy(dst_hbm.at[pl.ds(off, kp)], dst_v)
            for b in range(NB):
                pltpu.async_copy(y_hbm.at[src_v.at[b]], rows_v.at[b], sem.at[b])
            run_phase()
        plsc.subcore_barrier()

        for t in range(rpt // CH):
            sl = pl.ds(s * rpt + t * CH, CH)
            pltpu.sync_copy(acc_sh.at[sl], out_hbm.at[c, sl])

    return seg


def kernel(x, edge_index, W1_rel, b1, W1_root, W2_rel, b2, W2_root):
    n, d = x.shape
    e = edge_index.shape[1]
    assert d == D
    try:
        info = plsc.get_sparse_core_info()
        nc, ns = info.num_cores, info.num_subcores
    except Exception:
        nc, ns = 2, 16
    nw = nc * ns
    align = max(ns * CH, R)
    np_ = -(-(n + 1) // align) * align
    kq = -(-e // (nw * CH))
    kq = -(-kq // NB) * NB
    ep = nw * kq * CH

    src = edge_index[0]
    dst = edge_index[1]
    pad_dst = n + (jnp.arange(ep - e, dtype=jnp.int32) % (np_ - n))
    pad_src = jnp.arange(ep - e, dtype=jnp.int32) % n
    srcp = jnp.concatenate(
        [pad_src, src.astype(jnp.int32)]
    ).reshape(ep // CH, CH)
    dstp = jnp.concatenate([pad_dst, dst.astype(jnp.int32)]).reshape(ep // CH, CH)
    xp = jnp.pad(x, ((0, np_ - n), (0, 0)))
    b1r = b1.reshape(1, D)
    b2r = b2.reshape(1, D)

    seg = _make_segsum(np_, kq, nc, ns)
    y1, r1 = _dense_in(xp, W1_rel, b1r, W1_root, np_)
    acc1 = seg(y1, srcp, dstp)
    y2, r2 = _dense_mid(acc1, r1, W2_rel, b2r, W2_root, np_, n)
    acc2 = seg(y2, srcp, dstp)
    outp = _dense_out(acc2, r2, np_)
    return outp[:n]
